# baseline (speedup 1.0000x reference)
import functools

import jax
import jax.numpy as jnp
from jax import lax
from jax.experimental import pallas as pl
from jax.experimental.pallas import tpu as pltpu
from jax.experimental.pallas import tpu_sc as plsc

NUM_CORES = 2
NUM_SUBCORES = 16
NW = NUM_CORES * NUM_SUBCORES

BB = 128
L = 16


@functools.cache
def _build(B0, S, V, D):
    assert B0 % (NW * BB) == 0 and D % L == 0
    DB = D // 8
    nbb_per_w = B0 // (NW * BB)
    blk_tok = BB * S
    mesh = plsc.VectorSubcoreMesh(core_axis_name="c", subcore_axis_name="s")

    @functools.partial(
        pl.kernel,
        mesh=mesh,
        out_type=jax.ShapeDtypeStruct((S, DB, B0 // BB, 8, BB), jnp.float32),
        scratch_types=[
            pltpu.VMEM((blk_tok,), jnp.int32),
            pltpu.VMEM((S, BB), jnp.int32),
            pltpu.VMEM((2, BB, D), jnp.float32),
            pltpu.VMEM((2, DB, 8, BB), jnp.float32),
            pltpu.SemaphoreType.DMA,
            pltpu.SemaphoreType.DMA,
        ],
        compiler_params=pltpu.CompilerParams(
            use_tc_tiling_on_sc=False, needs_layout_passes=False
        ),
    )
    def gather_kernel(ids_hbm, table_hbm, out_hbm, idx_raw, idx_t, rows_v,
                      blk_v, sem_g, sem_w):
        wid = lax.axis_index("s") * NUM_CORES + lax.axis_index("c")

        def fire_gather(s, buf):
            pltpu.async_copy(
                table_hbm.at[idx_t.at[s]], rows_v.at[buf], sem_g
            )

        def drain_gather(buf):
            pltpu.make_async_copy(
                table_hbm.at[idx_t.at[0]], rows_v.at[buf], sem_g
            ).wait()

        def start_writes(bb, s, buf):
            for db in range(DB):
                pltpu.async_copy(
                    blk_v.at[buf, db], out_hbm.at[s, db, bb], sem_w
                )

        def drain_writes(buf):
            for db in range(DB):
                pltpu.make_async_copy(
                    blk_v.at[buf, db], out_hbm.at[0, db, 0], sem_w
                ).wait()

        def transpose_idx():
            def srow(s, carry):
                for k in range(BB // L):
                    bi = lax.broadcasted_iota(jnp.int32, (L,), 0) + (k * L)
                    v = plsc.load_gather(idx_raw, [bi * S + s])
                    idx_t[s, pl.ds(k * L, L)] = v
                return carry
            lax.fori_loop(0, S, srow, 0)

        def shuffle(buf):
            iota = lax.broadcasted_iota(jnp.int32, (L,), 0)
            for d in range(D):
                dvec = jnp.full((L,), d, jnp.int32)
                for k in range(BB // L):
                    v = plsc.load_gather(
                        rows_v.at[buf], [iota + (k * L), dvec]
                    )
                    blk_v[buf, d // 8, d % 8, pl.ds(k * L, L)] = v

        def block_body(i, carry):
            bb = wid * nbb_per_w + i
            pltpu.sync_copy(ids_hbm.at[pl.ds(bb * blk_tok, blk_tok)], idx_raw)
            transpose_idx()

            fire_gather(0, 0)

            def pair_body(p, carry2):
                for h in range(2):
                    s = 2 * p + h

                    @pl.when(s + 1 < S)
                    def _fire_next():
                        fire_gather(s + 1, 1 - h)

                    drain_gather(h)

                    @pl.when(i * S + s > 1)
                    def _reclaim():
                        drain_writes(h)

                    shuffle(h)
                    start_writes(bb, s, h)
                return carry2

            lax.fori_loop(0, S // 2, pair_body, 0)
            return carry

        lax.fori_loop(0, nbb_per_w, block_body, 0)

        for t in range(2):
            drain_writes(t)

    return gather_kernel


def kernel(token_ids, weights):
    B0, S = token_ids.shape
    V, D = weights.shape
    ids = token_ids.reshape(B0 * S).astype(jnp.int32)
    out5 = _build(B0, S, V, D)(ids, weights)
    return out5.transpose(2, 4, 0, 1, 3).reshape(B0, S, D)


# speedup vs baseline: 1.6257x; 1.6257x over previous
import functools

import jax
import jax.numpy as jnp
from jax import lax
from jax.experimental import pallas as pl
from jax.experimental.pallas import tpu as pltpu
from jax.experimental.pallas import tpu_sc as plsc

NUM_CORES = 2
NUM_SUBCORES = 16
NW = NUM_CORES * NUM_SUBCORES

SUB = 128
G = 512
N_SUB = G // SUB
NBUF = 3


@functools.cache
def _build(B, V, D):
    assert B % (NW * G) == 0
    b_per_w = B // NW
    n_gran = b_per_w // G
    mesh = plsc.VectorSubcoreMesh(core_axis_name="c", subcore_axis_name="s")

    @functools.partial(
        pl.kernel,
        mesh=mesh,
        out_type=jax.ShapeDtypeStruct((B, D), jnp.float32),
        scratch_types=[
            pltpu.VMEM((NBUF, G), jnp.int32),
            pltpu.VMEM((NBUF, G, D), jnp.float32),
            pltpu.SemaphoreType.DMA,
            pltpu.SemaphoreType.DMA,
            pltpu.SemaphoreType.DMA,
        ],
        compiler_params=pltpu.CompilerParams(use_tc_tiling_on_sc=False),
    )
    def gather_kernel(ids_hbm, table_hbm, out_hbm, idx_v, rows_v, sem_i,
                      sem_g, sem_w):
        wid = lax.axis_index("s") * NUM_CORES + lax.axis_index("c")
        base = wid * b_per_w

        def fire_gathers(gb, ib):
            for j in range(N_SUB):
                pltpu.async_copy(
                    table_hbm.at[idx_v.at[ib, pl.ds(j * SUB, SUB)]],
                    rows_v.at[gb, pl.ds(j * SUB, SUB)],
                    sem_g,
                )

        def drain_gathers(gb):
            for j in range(N_SUB):
                pltpu.make_async_copy(
                    table_hbm.at[idx_v.at[0, pl.ds(j * SUB, SUB)]],
                    rows_v.at[gb, pl.ds(j * SUB, SUB)],
                    sem_g,
                ).wait()

        def stage_idx(g, ib, async_=True):
            src = ids_hbm.at[pl.ds(base + g * G, G)]
            if async_:
                pltpu.async_copy(src, idx_v.at[ib], sem_i)
            else:
                pltpu.sync_copy(src, idx_v.at[ib])

        def drain_idx():
            pltpu.make_async_copy(
                ids_hbm.at[pl.ds(base, G)], idx_v.at[0], sem_i
            ).wait()

        def start_write(g, gb):
            pltpu.async_copy(
                rows_v.at[gb], out_hbm.at[pl.ds(base + g * G, G)], sem_w
            )

        def drain_write(gb):
            pltpu.make_async_copy(
                rows_v.at[gb], out_hbm.at[pl.ds(base, G)], sem_w
            ).wait()

        stage_idx(0, 0, async_=False)
        fire_gathers(0, 0)
        stage_idx(1, 1)

        def loop_body(g, carry):
            b = lax.rem(g, NBUF)
            nb = lax.rem(g + 1, NBUF)

            @pl.when(g + 1 < n_gran)
            def _fire_next():
                drain_idx()

                @pl.when(g >= 2)
                def _reclaim():
                    drain_write(nb)

                fire_gathers(nb, nb)

            drain_gathers(b)

            @pl.when(g + 2 < n_gran)
            def _prefetch_idx():
                stage_idx(g + 2, lax.rem(g + 2, NBUF))

            start_write(g, b)
            return carry

        lax.fori_loop(0, n_gran, loop_body, 0)

        for t in range(min(NBUF, n_gran)):
            drain_write(t)

    return gather_kernel


def kernel(token_ids, weights):
    B0, S = token_ids.shape
    V, D = weights.shape
    B = B0 * S
    ids = token_ids.reshape(B).astype(jnp.int32)
    out = _build(B, V, D)(ids, weights)
    return out.reshape(B0, S, D)
